# baseline (speedup 1.0000x reference)
import jax
import jax.numpy as jnp
from jax import lax
from jax.experimental import pallas as pl
from jax.experimental.pallas import tpu as pltpu
from jax.experimental.pallas import tpu_sc as plsc

BATCH = 4
SEQ = 8192
DIM = 64
TOTAL = BATCH * SEQ

NUM_CORES = 2
NUM_SUBCORES = 16
NUM_WORKERS = NUM_CORES * NUM_SUBCORES
PER_WORKER = TOTAL // NUM_WORKERS
CHUNK = 128
NCHUNK = PER_WORKER // CHUNK
NPAIR = NCHUNK // 2
CPAD = CHUNK + 5
LANES = 16


def _split_transpose(gbuf, cbuf, sbuf):
    @plsc.parallel_loop(0, CHUNK, unroll=1)
    def row(j):
        col = jnp.full((LANES,), j, dtype=jnp.int32)
        for c in range(DIM // LANES):
            rows = c * LANES + lax.iota(jnp.int32, LANES)
            plsc.store_scatter(cbuf, [rows, col], gbuf[j, pl.ds(c * LANES, LANES)])
            plsc.store_scatter(sbuf, [rows, col], gbuf[j, pl.ds(DIM + c * LANES, LANES)])


def _rope_kernel(pos_hbm, tab_hbm, cos_out, sin_out,
                 idx_v, g0, g1, c0, c1, s0, s1,
                 sem_g0, sem_g1, sem_c0, sem_c1, sem_s0, sem_s1):
    wid = lax.axis_index("s") * NUM_CORES + lax.axis_index("c")
    row0 = wid * NCHUNK
    b = wid // (NUM_WORKERS // BATCH)
    seq0 = (wid % (NUM_WORKERS // BATCH)) * PER_WORKER

    pltpu.sync_copy(pos_hbm.at[pl.ds(row0, NCHUNK)], idx_v)

    pltpu.async_copy(tab_hbm.at[idx_v.at[0]], g0, sem_g0)
    pltpu.async_copy(tab_hbm.at[idx_v.at[1]], g1, sem_g1)

    def slot(p, j, gb, cb, sb, sem_g, sem_c, sem_s):
        pltpu.make_async_copy(tab_hbm.at[idx_v.at[0]], gb, sem_g).wait()

        @pl.when(p > 0)
        def _():
            pltpu.make_async_copy(cb.at[:, pl.ds(0, CHUNK)], cos_out.at[b, :, pl.ds(seq0, CHUNK)], sem_c).wait()
            pltpu.make_async_copy(sb.at[:, pl.ds(0, CHUNK)], sin_out.at[b, :, pl.ds(seq0, CHUNK)], sem_s).wait()

        _split_transpose(gb, cb, sb)

        @pl.when(p < NPAIR - 1)
        def _():
            pltpu.async_copy(tab_hbm.at[idx_v.at[j + 2]], gb, sem_g)

        dst = pl.ds(seq0 + j * CHUNK, CHUNK)
        pltpu.async_copy(cb.at[:, pl.ds(0, CHUNK)], cos_out.at[b, :, dst], sem_c)
        pltpu.async_copy(sb.at[:, pl.ds(0, CHUNK)], sin_out.at[b, :, dst], sem_s)

    def body(p, carry):
        slot(p, 2 * p, g0, c0, s0, sem_g0, sem_c0, sem_s0)
        slot(p, 2 * p + 1, g1, c1, s1, sem_g1, sem_c1, sem_s1)
        return carry

    lax.fori_loop(0, NPAIR, body, 0)

    for cb, sb, sem_c, sem_s in ((c0, s0, sem_c0, sem_s0), (c1, s1, sem_c1, sem_s1)):
        pltpu.make_async_copy(cb.at[:, pl.ds(0, CHUNK)], cos_out.at[b, :, pl.ds(seq0, CHUNK)], sem_c).wait()
        pltpu.make_async_copy(sb.at[:, pl.ds(0, CHUNK)], sin_out.at[b, :, pl.ds(seq0, CHUNK)], sem_s).wait()


@jax.jit
def _rope_gather(pos2d, table):
    mesh = plsc.VectorSubcoreMesh(core_axis_name="c", subcore_axis_name="s")
    out_t = jax.ShapeDtypeStruct((BATCH, DIM, SEQ), jnp.float32)
    scratch = [
        pltpu.VMEM((NCHUNK, CHUNK), jnp.int32),
        pltpu.VMEM((CHUNK, 2 * DIM), jnp.float32),
        pltpu.VMEM((CHUNK, 2 * DIM), jnp.float32),
        pltpu.VMEM((DIM, CPAD), jnp.float32),
        pltpu.VMEM((DIM, CPAD), jnp.float32),
        pltpu.VMEM((DIM, CPAD), jnp.float32),
        pltpu.VMEM((DIM, CPAD), jnp.float32),
        pltpu.SemaphoreType.DMA,
        pltpu.SemaphoreType.DMA,
        pltpu.SemaphoreType.DMA,
        pltpu.SemaphoreType.DMA,
        pltpu.SemaphoreType.DMA,
        pltpu.SemaphoreType.DMA,
    ]
    return pl.kernel(
        _rope_kernel,
        out_type=(out_t, out_t),
        mesh=mesh,
        scratch_types=scratch,
        compiler_params=pltpu.CompilerParams(
            use_tc_tiling_on_sc=True, needs_layout_passes=False),
    )(pos2d, table)


def kernel(position_ids, cos_cache, sin_cache):
    pos2d = position_ids.astype(jnp.int32).reshape(TOTAL // CHUNK, CHUNK)
    table = jnp.concatenate([cos_cache, sin_cache], axis=1)
    cos_t, sin_t = _rope_gather(pos2d, table)
    return cos_t.transpose(0, 2, 1), sin_t.transpose(0, 2, 1)


# speedup vs baseline: 1.1979x; 1.1979x over previous
import jax
import jax.numpy as jnp
from jax import lax
from jax.experimental import pallas as pl
from jax.experimental.pallas import tpu as pltpu
from jax.experimental.pallas import tpu_sc as plsc

BATCH = 4
SEQ = 8192
DIM = 64
TOTAL = BATCH * SEQ

NUM_CORES = 2
NUM_SUBCORES = 16
NUM_WORKERS = NUM_CORES * NUM_SUBCORES
PER_WORKER = TOTAL // NUM_WORKERS
CHUNK = 128
NCHUNK = PER_WORKER // CHUNK


def _split_rows(gbuf, cbuf, sbuf):
    def row(r, carry):
        for c in range(DIM // 16):
            cbuf[r, pl.ds(c * 16, 16)] = gbuf[r, pl.ds(c * 16, 16)]
            sbuf[r, pl.ds(c * 16, 16)] = gbuf[r, pl.ds(DIM + c * 16, 16)]
        return carry
    lax.fori_loop(0, CHUNK, row, 0)


def _rope_kernel(pos_hbm, tab_hbm, cos_out, sin_out,
                 idx_v, g0, g1, c0, c1, s0, s1,
                 sem_g0, sem_g1, sem_c0, sem_c1, sem_s0, sem_s1):
    wid = lax.axis_index("s") * NUM_CORES + lax.axis_index("c")
    row0 = wid * NCHUNK
    base = wid * PER_WORKER

    gbuf = (g0, g1)
    cbuf = (c0, c1)
    sbuf = (s0, s1)
    sem_g = (sem_g0, sem_g1)
    sem_c = (sem_c0, sem_c1)
    sem_s = (sem_s0, sem_s1)

    pltpu.sync_copy(pos_hbm.at[pl.ds(row0, NCHUNK)], idx_v)

    gcp = [None] * NCHUNK
    wcp_c = [None, None]
    wcp_s = [None, None]
    gcp[0] = pltpu.async_copy(tab_hbm.at[idx_v.at[0]], gbuf[0], sem_g[0])
    for j in range(NCHUNK):
        s = j % 2
        if j + 1 < NCHUNK:
            gcp[j + 1] = pltpu.async_copy(
                tab_hbm.at[idx_v.at[j + 1]], gbuf[(j + 1) % 2], sem_g[(j + 1) % 2])
        gcp[j].wait()
        if j >= 2:
            wcp_c[s].wait()
            wcp_s[s].wait()
        _split_rows(gbuf[s], cbuf[s], sbuf[s])
        dst = pl.ds(base + j * CHUNK, CHUNK)
        wcp_c[s] = pltpu.async_copy(cbuf[s], cos_out.at[dst], sem_c[s])
        wcp_s[s] = pltpu.async_copy(sbuf[s], sin_out.at[dst], sem_s[s])
    for s in range(2):
        wcp_c[s].wait()
        wcp_s[s].wait()


@jax.jit
def _rope_gather(pos2d, table):
    mesh = plsc.VectorSubcoreMesh(core_axis_name="c", subcore_axis_name="s")
    out_t = jax.ShapeDtypeStruct((TOTAL, DIM), jnp.float32)
    scratch = [
        pltpu.VMEM((NCHUNK, CHUNK), jnp.int32),
        pltpu.VMEM((CHUNK, 2 * DIM), jnp.float32),
        pltpu.VMEM((CHUNK, 2 * DIM), jnp.float32),
        pltpu.VMEM((CHUNK, DIM), jnp.float32),
        pltpu.VMEM((CHUNK, DIM), jnp.float32),
        pltpu.VMEM((CHUNK, DIM), jnp.float32),
        pltpu.VMEM((CHUNK, DIM), jnp.float32),
        pltpu.SemaphoreType.DMA,
        pltpu.SemaphoreType.DMA,
        pltpu.SemaphoreType.DMA,
        pltpu.SemaphoreType.DMA,
        pltpu.SemaphoreType.DMA,
        pltpu.SemaphoreType.DMA,
    ]
    return pl.kernel(
        _rope_kernel,
        out_type=(out_t, out_t),
        mesh=mesh,
        scratch_types=scratch,
    )(pos2d, table)


def kernel(position_ids, cos_cache, sin_cache):
    pos2d = position_ids.astype(jnp.int32).reshape(TOTAL // CHUNK, CHUNK)
    table = jnp.concatenate([cos_cache, sin_cache], axis=1)
    cos, sin = _rope_gather(pos2d, table)
    shape = (*position_ids.shape, DIM)
    return cos.reshape(shape), sin.reshape(shape)
